# baseline (speedup 1.0000x reference)
import jax
import jax.numpy as jnp
from jax.experimental import pallas as pl

N_BARS = 256
B_STATIC = 64
EMB_D = 1024
B_BLK = 4


def _gather_body(idx_ref, emb_ref, out_ref):
    idx = idx_ref[...]
    cols = jax.lax.broadcasted_iota(jnp.int32, (N_BARS, N_BARS), 1)
    onehot = (idx == cols).astype(jnp.float32)
    out_ref[...] = jnp.dot(onehot, emb_ref[...],
                           preferred_element_type=jnp.float32)


def _bcast_body(emb_ref, out_ref):
    out_ref[...] = jnp.broadcast_to(emb_ref[...][None], out_ref.shape)


def kernel(num_bars, batch_size, embedding):
    shift = (num_bars - N_BARS) + (batch_size - B_STATIC)
    idx = jnp.clip(jnp.arange(N_BARS, dtype=jnp.int32) + shift, 0, N_BARS - 1)
    idx2 = idx.reshape(N_BARS, 1)

    gathered = pl.pallas_call(
        _gather_body,
        in_specs=[
            pl.BlockSpec((N_BARS, 1), lambda: (0, 0)),
            pl.BlockSpec((N_BARS, EMB_D), lambda: (0, 0)),
        ],
        out_specs=pl.BlockSpec((N_BARS, EMB_D), lambda: (0, 0)),
        out_shape=jax.ShapeDtypeStruct((N_BARS, EMB_D), jnp.float32),
    )(idx2, embedding)

    out = pl.pallas_call(
        _bcast_body,
        grid=(B_STATIC // B_BLK,),
        in_specs=[pl.BlockSpec((N_BARS, EMB_D), lambda i: (0, 0))],
        out_specs=pl.BlockSpec((B_BLK, N_BARS, EMB_D), lambda i: (i, 0, 0)),
        out_shape=jax.ShapeDtypeStruct((B_STATIC, N_BARS, EMB_D), jnp.float32),
    )(gathered)
    return out
